# baseline (speedup 1.0000x reference)
import functools

import jax
import jax.numpy as jnp
from jax import lax
from jax.experimental import pallas as pl
from jax.experimental.pallas import tpu as pltpu
from jax.experimental.pallas import tpu_sc as plsc

_N = 10000
_NP = 10240
_E = 320000
_B = 512
_DF = 128
_DIM = 256
_H = 128
_NT = 16
_EPT = _E // _NT
_K = 80
_NCHUNK = _EPT // _K
_RPT = _NP // _NT


def _mm(a, b):
    return jax.lax.dot(a, b, precision=jax.lax.Precision.HIGHEST)


def _node_mlp_body(x_ref, w_ref, b_ref, o0_ref, o1_ref):
    h = jnp.maximum(_mm(x_ref[...], w_ref[...]) + b_ref[...], 0.0)
    o0_ref[...] = h[:, :_H]
    o1_ref[...] = h[:, _H:]


def _node_mlp(xs, W0, b0):
    nb = 1024
    g = xs.shape[0] // nb
    return pl.pallas_call(
        _node_mlp_body,
        grid=(g,),
        in_specs=[
            pl.BlockSpec((nb, _DF), lambda i: (i, 0)),
            pl.BlockSpec((_DF, _DIM), lambda i: (0, 0)),
            pl.BlockSpec((1, _DIM), lambda i: (0, 0)),
        ],
        out_specs=[
            pl.BlockSpec((nb, _H), lambda i: (i, 0)),
            pl.BlockSpec((nb, _H), lambda i: (i, 0)),
        ],
        out_shape=[jax.ShapeDtypeStruct((xs.shape[0], _H), jnp.float32)] * 2,
    )(xs, W0, b0)


def _edge_mlp_body(ea_ref, w_ref, b_ref, o0_ref, o1_ref):
    e = _mm(ea_ref[...], w_ref[...]) + b_ref[...]
    o0_ref[...] = e[:, :_H]
    o1_ref[...] = e[:, _H:]


def _edge_mlp(eas, Wt, bt):
    eb = 4000
    g = eas.shape[0] // eb
    return pl.pallas_call(
        _edge_mlp_body,
        grid=(g,),
        in_specs=[
            pl.BlockSpec((eb, 16), lambda i: (i, 0)),
            pl.BlockSpec((16, _DIM), lambda i: (0, 0)),
            pl.BlockSpec((1, _DIM), lambda i: (0, 0)),
        ],
        out_specs=[
            pl.BlockSpec((eb, _H), lambda i: (i, 0)),
            pl.BlockSpec((eb, _H), lambda i: (i, 0)),
        ],
        out_shape=[jax.ShapeDtypeStruct((eas.shape[0], _H), jnp.float32)] * 2,
    )(eas, Wt, bt)


def _sc_gine_aggr(src_hbm, dst_hbm, h0_hbm, h1_hbm, e0_hbm, e1_hbm, z_hbm,
                  out0, out1, src0, src1, dst0, dst1, g0, g1, e0v, e1v,
                  aggr_sp, si0, si1, sg0, sg1, se0, se1):
    ci = lax.axis_index("c")
    si = lax.axis_index("s")
    rows = pl.ds(si * _RPT, _RPT)
    srcs = (src0, src1)
    dsts = (dst0, dst1)
    gaths = (g0, g1)
    eats = (e0v, e1v)
    sem_i = (si0, si1)
    sem_g = (sg0, sg1)
    sem_e = (se0, se1)

    def run(h_hbm, e_hbm, out_hbm):
        for m in range(3):
            pltpu.sync_copy(z_hbm.at[rows], aggr_sp.at[rows])
            plsc.subcore_barrier()
            base = m * _E + si * _EPT

            def start_idx(c, b):
                off = base + c * _K
                pltpu.async_copy(src_hbm.at[pl.ds(off, _K)],
                                 srcs[b], sem_i[b])
                pltpu.async_copy(dst_hbm.at[pl.ds(off, _K)],
                                 dsts[b], sem_i[b])

            def wait_idx(b):
                pltpu.make_async_copy(src_hbm.at[pl.ds(0, _K)],
                                      srcs[b], sem_i[b]).wait()
                pltpu.make_async_copy(dst_hbm.at[pl.ds(0, _K)],
                                      dsts[b], sem_i[b]).wait()

            def start_fetch(c, b):
                if m:
                    for j in range(_K // 16):
                        sl = pl.ds(j * 16, 16)
                        srcs[b][sl] = srcs[b][sl] + jnp.int32(m * _NP)
                pltpu.async_copy(h_hbm.at[srcs[b]], gaths[b], sem_g[b])
                pltpu.async_copy(e_hbm.at[pl.ds(base + c * _K, _K)],
                                 eats[b], sem_e[b])

            def wait_fetch(b):
                pltpu.make_async_copy(h_hbm.at[srcs[b]], gaths[b],
                                      sem_g[b]).wait()
                pltpu.make_async_copy(e_hbm.at[pl.ds(0, _K)], eats[b],
                                      sem_e[b]).wait()

            def compute(b):
                def row(r2, c2):
                    for j in range(_H // 16):
                        sl = pl.ds(j * 16, 16)
                        gaths[b][r2, sl] = jnp.maximum(
                            gaths[b][r2, sl] + eats[b][r2, sl], 0.0)
                    return c2
                lax.fori_loop(0, _K, row, 0, unroll=2)

            def step(c, b):
                wait_idx(b)
                start_fetch(c, b)
                b2 = 1 - b
                wait_fetch(b2)
                compute(b2)
                pltpu.sync_copy(gaths[b2], aggr_sp.at[dsts[b2]], add=True)

                @pl.when(c + 1 < _NCHUNK)
                def _():
                    start_idx(c + 1, b2)

            start_idx(0, 0)
            start_idx(1, 1)
            wait_idx(0)
            start_fetch(0, 0)

            def pair(p, carry):
                step(2 * p + 1, 1)
                step(2 * p + 2, 0)
                return carry

            lax.fori_loop(0, (_NCHUNK - 2) // 2, pair, 0)
            step(_NCHUNK - 1, 1)
            wait_fetch(1)
            compute(1)
            pltpu.sync_copy(gaths[1], aggr_sp.at[dsts[1]], add=True)
            plsc.subcore_barrier()
            pltpu.sync_copy(aggr_sp.at[rows],
                            out_hbm.at[pl.ds(m * _NP + si * _RPT, _RPT)])
            plsc.subcore_barrier()

    @pl.when(ci == 0)
    def _():
        run(h0_hbm, e0_hbm, out0)

    @pl.when(ci == 1)
    def _():
        run(h1_hbm, e1_hbm, out1)


def _sc_call(src, dst, h0, h1, e0, e1, zeros):
    f = pl.kernel(
        _sc_gine_aggr,
        mesh=plsc.VectorSubcoreMesh(core_axis_name="c", subcore_axis_name="s"),
        out_type=[jax.ShapeDtypeStruct((3 * _NP, _H), jnp.float32)] * 2,
        scratch_types=[
            pltpu.VMEM((_K,), jnp.int32),
            pltpu.VMEM((_K,), jnp.int32),
            pltpu.VMEM((_K,), jnp.int32),
            pltpu.VMEM((_K,), jnp.int32),
            pltpu.VMEM((_K, _H), jnp.float32),
            pltpu.VMEM((_K, _H), jnp.float32),
            pltpu.VMEM((_K, _H), jnp.float32),
            pltpu.VMEM((_K, _H), jnp.float32),
            pltpu.VMEM_SHARED((_NP, _H), jnp.float32),
            pltpu.SemaphoreType.DMA,
            pltpu.SemaphoreType.DMA,
            pltpu.SemaphoreType.DMA,
            pltpu.SemaphoreType.DMA,
            pltpu.SemaphoreType.DMA,
            pltpu.SemaphoreType.DMA,
        ],
    )
    return f(src, dst, h0, h1, e0, e1, zeros)


def _pool_body(hc0, hc1, ac0, ac1, bat, w1a, b1a, w1b, b1b, pooled):
    nb = pl.program_id(1)
    x = jnp.concatenate(
        [hc0[...] + ac0[...], hc1[...] + ac1[...]], axis=1)
    t = jnp.maximum(_mm(x, w1a[...]) + b1a[...], 0.0)
    o = jnp.maximum(_mm(t, w1b[...]) + b1b[...], 0.0)
    seg = bat[0, 0, :]
    oh = (lax.broadcasted_iota(jnp.int32, (_B, seg.shape[0]), 0)
          == seg[None, :]).astype(jnp.float32)
    acc = _mm(oh, o)

    @pl.when(nb == 0)
    def _():
        pooled[...] = acc[None]

    @pl.when(nb != 0)
    def _():
        pooled[...] += acc[None]


def _mlp_pool(h0, h1, a0, a1, bat, W1a, b1a, W1b, b1b):
    nb = 1024
    nblk = _NP // nb
    return pl.pallas_call(
        _pool_body,
        grid=(3, nblk),
        in_specs=[
            pl.BlockSpec((nb, _H), lambda m, i: (m * nblk + i, 0)),
            pl.BlockSpec((nb, _H), lambda m, i: (m * nblk + i, 0)),
            pl.BlockSpec((nb, _H), lambda m, i: (m * nblk + i, 0)),
            pl.BlockSpec((nb, _H), lambda m, i: (m * nblk + i, 0)),
            pl.BlockSpec((1, 1, nb), lambda m, i: (m * nblk + i, 0, 0)),
            pl.BlockSpec((_DIM, 2 * _DIM), lambda m, i: (0, 0)),
            pl.BlockSpec((1, 2 * _DIM), lambda m, i: (0, 0)),
            pl.BlockSpec((2 * _DIM, _DIM), lambda m, i: (0, 0)),
            pl.BlockSpec((1, _DIM), lambda m, i: (0, 0)),
        ],
        out_specs=pl.BlockSpec((1, _B, _DIM), lambda m, i: (m, 0, 0)),
        out_shape=jax.ShapeDtypeStruct((3, _B, _DIM), jnp.float32),
    )(h0, h1, a0, a1, bat, W1a, b1a, W1b, b1b)


def _head_body(pool_ref, scal_ref, wge_ref, bge_ref, w2a_ref, b2a_ref,
               w2b_ref, b2b_ref, wf1_ref, bf1_ref, wf2m_ref, wf2t_ref,
               bf2_ref, wf3_ref, bf3_ref, out_ref):
    sc = scal_ref[...]
    x0 = pool_ref[0] * sc[:, 0:1]
    x1 = pool_ref[1] * sc[:, 1:2]
    x2 = pool_ref[2] * sc[:, 2:3]
    w = wge_ref[...]
    bg = bge_ref[...]
    ehi = sc[:, 4:5] * w + bg
    eh13 = sc[:, 5:6] * w + bg
    eh23 = sc[:, 6:7] * w + bg
    r = lambda v: jnp.maximum(v, 0.0)
    a0 = r(x1 + ehi) + r(x2 + eh13)
    a1 = r(x0 + ehi) + r(x2 + eh23)
    a2 = r(x1 + eh23) + r(x0 + eh13)
    g = jnp.concatenate([x0 + a0, x1 + a1, x2 + a2], axis=0)
    t = r(_mm(g, w2a_ref[...]) + b2a_ref[...])
    y = r(_mm(t, w2b_ref[...]) + b2b_ref[...])
    fp = y[0:_B] + y[_B:2 * _B] + y[2 * _B:3 * _B]
    h = r(_mm(fp, wf1_ref[...]) + bf1_ref[...])
    temp = 10.0 * sc[:, 3:4]
    h2 = r(_mm(h, wf2m_ref[...]) + temp * wf2t_ref[...] + bf2_ref[...])
    out_ref[...] = _mm(h2, wf3_ref[...]) + bf3_ref[...]


def _head(pooled, scal, Wge, bge, W2a, b2a, W2b, b2b,
          Wf1, bf1, Wf2m, Wf2t, bf2, Wf3, bf3):
    return pl.pallas_call(
        _head_body,
        out_shape=jax.ShapeDtypeStruct((_B, 1), jnp.float32),
    )(pooled, scal, Wge, bge, W2a, b2a, W2b, b2b,
      Wf1, bf1, Wf2m, Wf2t, bf2, Wf3, bf3)


def kernel(x_1, edge_index_1, edge_attr_1, batch_1, x_2, edge_index_2,
           edge_attr_2, batch_2, x_3, edge_index_3, edge_attr_3, batch_3,
           ratio_1, ratio_2, ratio_3, T, h_inter, h_intra_1, h_inter_1_3,
           h_inter_2_3, W0, b0, Wt, bt, W1a, b1a, W1b, b1b, Wge, bge,
           W2a, b2a, W2b, b2b, Wf1, bf1, Wf2, bf2, Wf3, bf3):
    pad = ((0, _NP - _N), (0, 0))
    xs = jnp.concatenate(
        [jnp.pad(x_1, pad), jnp.pad(x_2, pad), jnp.pad(x_3, pad)], axis=0)
    eas = jnp.concatenate([edge_attr_1, edge_attr_2, edge_attr_3], axis=0)
    src = jnp.concatenate([edge_index_1[0], edge_index_2[0], edge_index_3[0]])
    dst = jnp.concatenate([edge_index_1[1], edge_index_2[1], edge_index_3[1]])
    bpad = jnp.full((_NP - _N,), _B, jnp.int32)
    bat = jnp.concatenate(
        [batch_1, bpad, batch_2, bpad, batch_3, bpad]).reshape(30, 1, 1024)

    h0, h1 = _node_mlp(xs, W0, b0.reshape(1, -1))
    e0, e1 = _edge_mlp(eas, Wt, bt.reshape(1, -1))
    zeros = jnp.zeros((_NP, _H), jnp.float32)
    a0, a1 = _sc_call(src, dst, h0, h1, e0, e1, zeros)
    pooled = _mlp_pool(h0, h1, a0, a1, bat, W1a, b1a.reshape(1, -1),
                       W1b, b1b.reshape(1, -1))
    scal = jnp.stack(
        [ratio_1, ratio_2, ratio_3, T, h_inter, h_inter_1_3, h_inter_2_3],
        axis=1)
    return _head(pooled, scal, Wge, bge.reshape(1, -1),
                 W2a, b2a.reshape(1, -1), W2b, b2b.reshape(1, -1),
                 Wf1, bf1.reshape(1, -1), Wf2[:_DIM], Wf2[_DIM:_DIM + 1],
                 bf2.reshape(1, -1), Wf3, bf3.reshape(1, -1))


# speedup vs baseline: 1.6731x; 1.6731x over previous
import functools

import jax
import jax.numpy as jnp
from jax import lax
from jax.experimental import pallas as pl
from jax.experimental.pallas import tpu as pltpu
from jax.experimental.pallas import tpu_sc as plsc

_N = 10000
_NP = 10240
_E = 320000
_B = 512
_DF = 128
_DIM = 256
_H = 128
_NT = 16
_EPT = _E // _NT
_K = 80
_NCHUNK = _EPT // _K
_RPT = _NP // _NT


def _mm(a, b):
    return jax.lax.dot(a, b, precision=jax.lax.Precision.HIGHEST)


def _node_mlp_body(x_ref, w_ref, b_ref, o0_ref, o1_ref):
    h = jnp.maximum(_mm(x_ref[...], w_ref[...]) + b_ref[...], 0.0)
    o0_ref[...] = h[:, :_H]
    o1_ref[...] = h[:, _H:]


def _node_mlp(xs, W0, b0):
    nb = 1024
    g = xs.shape[0] // nb
    return pl.pallas_call(
        _node_mlp_body,
        grid=(g,),
        in_specs=[
            pl.BlockSpec((nb, _DF), lambda i: (i, 0)),
            pl.BlockSpec((_DF, _DIM), lambda i: (0, 0)),
            pl.BlockSpec((1, _DIM), lambda i: (0, 0)),
        ],
        out_specs=[
            pl.BlockSpec((nb, _H), lambda i: (i, 0)),
            pl.BlockSpec((nb, _H), lambda i: (i, 0)),
        ],
        out_shape=[jax.ShapeDtypeStruct((xs.shape[0], _H), jnp.float32)] * 2,
    )(xs, W0, b0)


def _edge_mlp_body(ea_ref, w_ref, b_ref, o0_ref, o1_ref):
    e = _mm(ea_ref[...], w_ref[...]) + b_ref[...]
    o0_ref[...] = e[:, :_H]
    o1_ref[...] = e[:, _H:]


def _edge_mlp(eas, Wt, bt):
    eb = 4000
    g = eas.shape[0] // eb
    return pl.pallas_call(
        _edge_mlp_body,
        grid=(g,),
        in_specs=[
            pl.BlockSpec((eb, 16), lambda i: (i, 0)),
            pl.BlockSpec((16, _DIM), lambda i: (0, 0)),
            pl.BlockSpec((1, _DIM), lambda i: (0, 0)),
        ],
        out_specs=[
            pl.BlockSpec((eb, _H), lambda i: (i, 0)),
            pl.BlockSpec((eb, _H), lambda i: (i, 0)),
        ],
        out_shape=[jax.ShapeDtypeStruct((eas.shape[0], _H), jnp.float32)] * 2,
    )(eas, Wt, bt)


def _sc_gine_aggr(src_hbm, dst_hbm, h0_hbm, h1_hbm, e0_hbm, e1_hbm, z_hbm,
                  out0, out1, src0, src1, dst0, dst1, g0, g1, e0v, e1v,
                  aggr_sp, si0, si1, sg0, sg1, se0, se1):
    ci = lax.axis_index("c")
    si = lax.axis_index("s")
    rows = pl.ds(si * _RPT, _RPT)
    srcs = (src0, src1)
    dsts = (dst0, dst1)
    gaths = (g0, g1)
    eats = (e0v, e1v)
    sem_i = (si0, si1)
    sem_g = (sg0, sg1)
    sem_e = (se0, se1)

    def run(h_hbm, e_hbm, out_hbm):
        for m in range(3):
            pltpu.sync_copy(z_hbm.at[rows], aggr_sp.at[rows])
            plsc.subcore_barrier()
            base = m * _E + si * _EPT

            def start_idx(c, b):
                off = base + c * _K
                pltpu.async_copy(src_hbm.at[pl.ds(off, _K)],
                                 srcs[b], sem_i[b])
                pltpu.async_copy(dst_hbm.at[pl.ds(off, _K)],
                                 dsts[b], sem_i[b])

            def wait_idx(b):
                pltpu.make_async_copy(src_hbm.at[pl.ds(0, _K)],
                                      srcs[b], sem_i[b]).wait()
                pltpu.make_async_copy(dst_hbm.at[pl.ds(0, _K)],
                                      dsts[b], sem_i[b]).wait()

            def start_fetch(c, b):
                if m:
                    for j in range(_K // 16):
                        sl = pl.ds(j * 16, 16)
                        srcs[b][sl] = srcs[b][sl] + jnp.int32(m * _NP)
                pltpu.async_copy(h_hbm.at[srcs[b]], gaths[b], sem_g[b])
                pltpu.async_copy(e_hbm.at[pl.ds(base + c * _K, _K)],
                                 eats[b], sem_e[b])

            def wait_fetch(b):
                pltpu.make_async_copy(h_hbm.at[srcs[b]], gaths[b],
                                      sem_g[b]).wait()
                pltpu.make_async_copy(e_hbm.at[pl.ds(0, _K)], eats[b],
                                      sem_e[b]).wait()

            def compute(b):
                @plsc.parallel_loop(0, _K, unroll=4)
                def _(r2):
                    for j in range(_H // 16):
                        sl = pl.ds(j * 16, 16)
                        gaths[b][r2, sl] = jnp.maximum(
                            gaths[b][r2, sl] + eats[b][r2, sl], 0.0)

            def step(c, b):
                wait_idx(b)
                start_fetch(c, b)
                b2 = 1 - b
                wait_fetch(b2)
                compute(b2)
                pltpu.sync_copy(gaths[b2], aggr_sp.at[dsts[b2]], add=True)

                @pl.when(c + 1 < _NCHUNK)
                def _():
                    start_idx(c + 1, b2)

            start_idx(0, 0)
            start_idx(1, 1)
            wait_idx(0)
            start_fetch(0, 0)

            def pair(p, carry):
                step(2 * p + 1, 1)
                step(2 * p + 2, 0)
                return carry

            lax.fori_loop(0, (_NCHUNK - 2) // 2, pair, 0)
            step(_NCHUNK - 1, 1)
            wait_fetch(1)
            compute(1)
            pltpu.sync_copy(gaths[1], aggr_sp.at[dsts[1]], add=True)
            plsc.subcore_barrier()
            pltpu.sync_copy(aggr_sp.at[rows],
                            out_hbm.at[pl.ds(m * _NP + si * _RPT, _RPT)])
            plsc.subcore_barrier()

    @pl.when(ci == 0)
    def _():
        run(h0_hbm, e0_hbm, out0)

    @pl.when(ci == 1)
    def _():
        run(h1_hbm, e1_hbm, out1)


def _sc_call(src, dst, h0, h1, e0, e1, zeros):
    f = pl.kernel(
        _sc_gine_aggr,
        mesh=plsc.VectorSubcoreMesh(core_axis_name="c", subcore_axis_name="s"),
        out_type=[jax.ShapeDtypeStruct((3 * _NP, _H), jnp.float32)] * 2,
        scratch_types=[
            pltpu.VMEM((_K,), jnp.int32),
            pltpu.VMEM((_K,), jnp.int32),
            pltpu.VMEM((_K,), jnp.int32),
            pltpu.VMEM((_K,), jnp.int32),
            pltpu.VMEM((_K, _H), jnp.float32),
            pltpu.VMEM((_K, _H), jnp.float32),
            pltpu.VMEM((_K, _H), jnp.float32),
            pltpu.VMEM((_K, _H), jnp.float32),
            pltpu.VMEM_SHARED((_NP, _H), jnp.float32),
            pltpu.SemaphoreType.DMA,
            pltpu.SemaphoreType.DMA,
            pltpu.SemaphoreType.DMA,
            pltpu.SemaphoreType.DMA,
            pltpu.SemaphoreType.DMA,
            pltpu.SemaphoreType.DMA,
        ],
    )
    return f(src, dst, h0, h1, e0, e1, zeros)


def _pool_body(hc0, hc1, ac0, ac1, bat, w1a, b1a, w1b, b1b, pooled):
    nb = pl.program_id(1)
    x = jnp.concatenate(
        [hc0[...] + ac0[...], hc1[...] + ac1[...]], axis=1)
    t = jnp.maximum(_mm(x, w1a[...]) + b1a[...], 0.0)
    o = jnp.maximum(_mm(t, w1b[...]) + b1b[...], 0.0)
    seg = bat[0, 0, :]
    oh = (lax.broadcasted_iota(jnp.int32, (_B, seg.shape[0]), 0)
          == seg[None, :]).astype(jnp.float32)
    acc = _mm(oh, o)

    @pl.when(nb == 0)
    def _():
        pooled[...] = acc[None]

    @pl.when(nb != 0)
    def _():
        pooled[...] += acc[None]


def _mlp_pool(h0, h1, a0, a1, bat, W1a, b1a, W1b, b1b):
    nb = 1024
    nblk = _NP // nb
    return pl.pallas_call(
        _pool_body,
        grid=(3, nblk),
        in_specs=[
            pl.BlockSpec((nb, _H), lambda m, i: (m * nblk + i, 0)),
            pl.BlockSpec((nb, _H), lambda m, i: (m * nblk + i, 0)),
            pl.BlockSpec((nb, _H), lambda m, i: (m * nblk + i, 0)),
            pl.BlockSpec((nb, _H), lambda m, i: (m * nblk + i, 0)),
            pl.BlockSpec((1, 1, nb), lambda m, i: (m * nblk + i, 0, 0)),
            pl.BlockSpec((_DIM, 2 * _DIM), lambda m, i: (0, 0)),
            pl.BlockSpec((1, 2 * _DIM), lambda m, i: (0, 0)),
            pl.BlockSpec((2 * _DIM, _DIM), lambda m, i: (0, 0)),
            pl.BlockSpec((1, _DIM), lambda m, i: (0, 0)),
        ],
        out_specs=pl.BlockSpec((1, _B, _DIM), lambda m, i: (m, 0, 0)),
        out_shape=jax.ShapeDtypeStruct((3, _B, _DIM), jnp.float32),
    )(h0, h1, a0, a1, bat, W1a, b1a, W1b, b1b)


def _head_body(pool_ref, scal_ref, wge_ref, bge_ref, w2a_ref, b2a_ref,
               w2b_ref, b2b_ref, wf1_ref, bf1_ref, wf2m_ref, wf2t_ref,
               bf2_ref, wf3_ref, bf3_ref, out_ref):
    sc = scal_ref[...]
    x0 = pool_ref[0] * sc[:, 0:1]
    x1 = pool_ref[1] * sc[:, 1:2]
    x2 = pool_ref[2] * sc[:, 2:3]
    w = wge_ref[...]
    bg = bge_ref[...]
    ehi = sc[:, 4:5] * w + bg
    eh13 = sc[:, 5:6] * w + bg
    eh23 = sc[:, 6:7] * w + bg
    r = lambda v: jnp.maximum(v, 0.0)
    a0 = r(x1 + ehi) + r(x2 + eh13)
    a1 = r(x0 + ehi) + r(x2 + eh23)
    a2 = r(x1 + eh23) + r(x0 + eh13)
    g = jnp.concatenate([x0 + a0, x1 + a1, x2 + a2], axis=0)
    t = r(_mm(g, w2a_ref[...]) + b2a_ref[...])
    y = r(_mm(t, w2b_ref[...]) + b2b_ref[...])
    fp = y[0:_B] + y[_B:2 * _B] + y[2 * _B:3 * _B]
    h = r(_mm(fp, wf1_ref[...]) + bf1_ref[...])
    temp = 10.0 * sc[:, 3:4]
    h2 = r(_mm(h, wf2m_ref[...]) + temp * wf2t_ref[...] + bf2_ref[...])
    out_ref[...] = _mm(h2, wf3_ref[...]) + bf3_ref[...]


def _head(pooled, scal, Wge, bge, W2a, b2a, W2b, b2b,
          Wf1, bf1, Wf2m, Wf2t, bf2, Wf3, bf3):
    return pl.pallas_call(
        _head_body,
        out_shape=jax.ShapeDtypeStruct((_B, 1), jnp.float32),
    )(pooled, scal, Wge, bge, W2a, b2a, W2b, b2b,
      Wf1, bf1, Wf2m, Wf2t, bf2, Wf3, bf3)


def kernel(x_1, edge_index_1, edge_attr_1, batch_1, x_2, edge_index_2,
           edge_attr_2, batch_2, x_3, edge_index_3, edge_attr_3, batch_3,
           ratio_1, ratio_2, ratio_3, T, h_inter, h_intra_1, h_inter_1_3,
           h_inter_2_3, W0, b0, Wt, bt, W1a, b1a, W1b, b1b, Wge, bge,
           W2a, b2a, W2b, b2b, Wf1, bf1, Wf2, bf2, Wf3, bf3):
    pad = ((0, _NP - _N), (0, 0))
    xs = jnp.concatenate(
        [jnp.pad(x_1, pad), jnp.pad(x_2, pad), jnp.pad(x_3, pad)], axis=0)
    eas = jnp.concatenate([edge_attr_1, edge_attr_2, edge_attr_3], axis=0)
    src = jnp.concatenate([edge_index_1[0], edge_index_2[0], edge_index_3[0]])
    dst = jnp.concatenate([edge_index_1[1], edge_index_2[1], edge_index_3[1]])
    bpad = jnp.full((_NP - _N,), _B, jnp.int32)
    bat = jnp.concatenate(
        [batch_1, bpad, batch_2, bpad, batch_3, bpad]).reshape(30, 1, 1024)

    h0, h1 = _node_mlp(xs, W0, b0.reshape(1, -1))
    e0, e1 = _edge_mlp(eas, Wt, bt.reshape(1, -1))
    zeros = jnp.zeros((_NP, _H), jnp.float32)
    a0, a1 = _sc_call(src, dst, h0, h1, e0, e1, zeros)
    pooled = _mlp_pool(h0, h1, a0, a1, bat, W1a, b1a.reshape(1, -1),
                       W1b, b1b.reshape(1, -1))
    scal = jnp.stack(
        [ratio_1, ratio_2, ratio_3, T, h_inter, h_inter_1_3, h_inter_2_3],
        axis=1)
    return _head(pooled, scal, Wge, bge.reshape(1, -1),
                 W2a, b2a.reshape(1, -1), W2b, b2b.reshape(1, -1),
                 Wf1, bf1.reshape(1, -1), Wf2[:_DIM], Wf2[_DIM:_DIM + 1],
                 bf2.reshape(1, -1), Wf3, bf3.reshape(1, -1))
